# baseline (speedup 1.0000x reference)
import functools

import jax
import jax.numpy as jnp
from jax import lax
from jax.experimental import pallas as pl
from jax.experimental.pallas import tpu as pltpu

F32 = jnp.float32
N_NODES = 10000
N_EDGES = 320000
HID = 128
EDGE_IN = 16

_PREP_BLK = 2000


def _prep_body(nodes_ref, wne_ref, bne_ref, w1b_ref, w1c_ref,
               ne_ref, a_ref, b_ref):
    ne = jnp.dot(nodes_ref[...], wne_ref[...], preferred_element_type=F32)
    ne = ne + bne_ref[...]
    ne_ref[...] = ne
    a_ref[...] = jnp.dot(ne, w1b_ref[...], preferred_element_type=F32)
    b_ref[...] = jnp.dot(ne, w1c_ref[...], preferred_element_type=F32)


def _tc_prep(nodes, W_ne, b_ne, W1b, W1c):
    n = nodes.shape[0]
    grid = (n // _PREP_BLK,)
    blk = lambda i: (i, 0)
    full = lambda i: (0, 0)
    return pl.pallas_call(
        _prep_body,
        grid=grid,
        in_specs=[
            pl.BlockSpec((_PREP_BLK, HID), blk),
            pl.BlockSpec((HID, HID), full),
            pl.BlockSpec((1, HID), full),
            pl.BlockSpec((HID, HID), full),
            pl.BlockSpec((HID, HID), full),
        ],
        out_specs=[
            pl.BlockSpec((_PREP_BLK, HID), blk),
            pl.BlockSpec((_PREP_BLK, HID), blk),
            pl.BlockSpec((_PREP_BLK, HID), blk),
        ],
        out_shape=[jax.ShapeDtypeStruct((n, HID), F32)] * 3,
    )(nodes, W_ne, b_ne.reshape(1, HID), W1b, W1c)


_EDGE_BLK = 8000


def _edge_body(edges_ref, g_ref, wee_ref, w1a_ref, bee_ref, be1_ref,
               gterm_ref, we2_ref, be2_ref, out_ref):
    wc = jnp.dot(wee_ref[...], w1a_ref[...], preferred_element_type=F32)
    c = (jnp.dot(bee_ref[...], w1a_ref[...], preferred_element_type=F32)
         + be1_ref[...] + gterm_ref[...])
    pre = jnp.dot(edges_ref[...], wc, preferred_element_type=F32) + g_ref[...] + c
    h = jnp.maximum(pre, 0.0)
    out_ref[...] = jnp.dot(h, we2_ref[...], preferred_element_type=F32) + be2_ref[...]


def _tc_edge(edges, G, base_blk, W_ee, W1a, b_ee, b_e1, gterm, W_e2, b_e2):
    e = G.shape[0]
    grid = (e // _EDGE_BLK,)
    blk = lambda i: (i, 0)
    eblk = lambda i: (i + base_blk, 0)
    full = lambda i: (0, 0)
    return pl.pallas_call(
        _edge_body,
        grid=grid,
        in_specs=[
            pl.BlockSpec((_EDGE_BLK, EDGE_IN), eblk),
            pl.BlockSpec((_EDGE_BLK, HID), blk),
            pl.BlockSpec((EDGE_IN, HID), full),
            pl.BlockSpec((HID, HID), full),
            pl.BlockSpec((1, HID), full),
            pl.BlockSpec((1, HID), full),
            pl.BlockSpec((1, HID), full),
            pl.BlockSpec((HID, HID), full),
            pl.BlockSpec((1, HID), full),
        ],
        out_specs=pl.BlockSpec((_EDGE_BLK, HID), blk),
        out_shape=jax.ShapeDtypeStruct((e, HID), F32),
    )(edges, G, W_ee, W1a, b_ee.reshape(1, HID), b_e1.reshape(1, HID),
      gterm, W_e2, b_e2.reshape(1, HID))


_NODE_BLK = 2000


def _node_body(ne_ref, sent0_ref, sent1_ref, recv0_ref, recv1_ref,
               wn1a_ref, wn1b_ref, wn1c_ref,
               cn_ref, wn2_ref, bn2_ref, bge_ref, wg1_ref, bg1_ref,
               wg2_ref, bg2_ref, nodes_u_ref, glob_ref, accn_ref, acce_ref):
    i = pl.program_id(0)
    nblk = pl.num_programs(0)
    sent = sent0_ref[...] + sent1_ref[...]
    recv = recv0_ref[...] + recv1_ref[...]
    pre = (jnp.dot(ne_ref[...], wn1a_ref[...], preferred_element_type=F32)
           + jnp.dot(sent, wn1b_ref[...], preferred_element_type=F32)
           + jnp.dot(recv, wn1c_ref[...], preferred_element_type=F32)
           + cn_ref[...])
    nu = (jnp.dot(jnp.maximum(pre, 0.0), wn2_ref[...], preferred_element_type=F32)
          + bn2_ref[...])
    nodes_u_ref[...] = nu
    blk_n = jnp.sum(nu, axis=0, keepdims=True)
    blk_e = jnp.sum(sent, axis=0, keepdims=True)

    @pl.when(i == 0)
    def _init():
        accn_ref[...] = blk_n
        acce_ref[...] = blk_e

    @pl.when(i > 0)
    def _acc():
        accn_ref[...] += blk_n
        acce_ref[...] += blk_e

    @pl.when(i == nblk - 1)
    def _glob():
        gf = jnp.concatenate([accn_ref[...], acce_ref[...], bge_ref[...]], axis=1)
        gh = jnp.maximum(
            jnp.dot(gf, wg1_ref[...], preferred_element_type=F32) + bg1_ref[...], 0.0)
        glob_ref[...] = (jnp.dot(gh, wg2_ref[...], preferred_element_type=F32)
                         + bg2_ref[...])


def _tc_node(nodes_e, sent0, sent1, recv0, recv1,
             Wn1a, Wn1b, Wn1c, cn, W_n2, b_n2,
             b_ge, W_g1, b_g1, W_g2, b_g2):
    n = nodes_e.shape[0]
    grid = (n // _NODE_BLK,)
    blk = lambda i: (i, 0)
    full = lambda i: (0, 0)
    return pl.pallas_call(
        _node_body,
        grid=grid,
        in_specs=[
            pl.BlockSpec((_NODE_BLK, HID), blk),
            pl.BlockSpec((_NODE_BLK, HID), blk),
            pl.BlockSpec((_NODE_BLK, HID), blk),
            pl.BlockSpec((_NODE_BLK, HID), blk),
            pl.BlockSpec((_NODE_BLK, HID), blk),
            pl.BlockSpec((HID, HID), full),
            pl.BlockSpec((HID, HID), full),
            pl.BlockSpec((HID, HID), full),
            pl.BlockSpec((1, HID), full),
            pl.BlockSpec((HID, HID), full),
            pl.BlockSpec((1, HID), full),
            pl.BlockSpec((1, HID), full),
            pl.BlockSpec((3 * HID, HID), full),
            pl.BlockSpec((1, HID), full),
            pl.BlockSpec((HID, 3), full),
            pl.BlockSpec((1, 3), full),
        ],
        out_specs=[
            pl.BlockSpec((_NODE_BLK, HID), blk),
            pl.BlockSpec((1, 3), full),
        ],
        out_shape=[
            jax.ShapeDtypeStruct((n, HID), F32),
            jax.ShapeDtypeStruct((1, 3), F32),
        ],
        scratch_shapes=[pltpu.VMEM((1, HID), F32), pltpu.VMEM((1, HID), F32)],
    )(nodes_e, sent0, sent1, recv0, recv1, Wn1a, Wn1b, Wn1c, cn, W_n2,
      b_n2.reshape(1, HID), b_ge.reshape(1, HID), W_g1, b_g1.reshape(1, HID),
      W_g2, b_g2.reshape(1, 3))


_NC = 2
_NS = 16
_NW = _NC * _NS
_CH = 80
_GCH = N_EDGES // _NW // _CH
_SCH = N_EDGES // _NS // _CH
_PIECE = 80
_NPIECE = N_NODES // _PIECE


def _sc_gather_add(A, B, senders2d, receivers2d):
    from jax.experimental.pallas import tpu_sc as plsc

    mesh = plsc.VectorSubcoreMesh(core_axis_name="c", subcore_axis_name="s",
                                  num_cores=_NC, num_subcores=_NS)

    D = 4
    gch = senders2d.shape[1]

    @functools.partial(
        pl.kernel, mesh=mesh,
        out_type=jax.ShapeDtypeStruct((_NW * gch * _CH, HID), F32),
        scratch_types=[
            pltpu.VMEM((gch, _CH), jnp.int32),
            pltpu.VMEM((gch, _CH), jnp.int32),
            pltpu.VMEM((D, _CH, HID), F32),
            pltpu.VMEM((D, _CH, HID), F32),
            [pltpu.SemaphoreType.DMA] * D,
            [pltpu.SemaphoreType.DMA] * D,
        ],
    )
    def gather_k(a_hbm, b_hbm, s_hbm, r_hbm, out_hbm,
                 sidx, ridx, rows_a, rows_b, semg, semo):
        wid = lax.axis_index("s") * _NC + lax.axis_index("c")
        pltpu.sync_copy(s_hbm.at[wid], sidx)
        pltpu.sync_copy(r_hbm.at[wid], ridx)
        ebase = pl.multiple_of(wid * (gch * _CH), _CH)

        def issue(j, b):
            @pl.when(j >= D)
            def _dr():
                pltpu.make_async_copy(rows_a.at[b],
                                      out_hbm.at[pl.ds(0, _CH), :],
                                      semo[b]).wait()

            pltpu.async_copy(a_hbm.at[sidx.at[j]], rows_a.at[b], semg[b])
            pltpu.async_copy(b_hbm.at[ridx.at[j]], rows_b.at[b], semg[b])

        def process(j, b):
            pltpu.make_async_copy(a_hbm.at[sidx.at[0]], rows_a.at[b],
                                  semg[b]).wait()
            pltpu.make_async_copy(b_hbm.at[ridx.at[0]], rows_b.at[b],
                                  semg[b]).wait()
            ra = rows_a.at[b]
            rb = rows_b.at[b]

            def addrow(r, c2):
                for k in range(HID // 16):
                    plsc.addupdate(ra.at[r, pl.ds(k * 16, 16)],
                                   rb[r, pl.ds(k * 16, 16)])
                return c2

            lax.fori_loop(0, _CH, addrow, 0, unroll=4)
            off = pl.multiple_of(ebase + j * _CH, _CH)
            pltpu.async_copy(ra, out_hbm.at[pl.ds(off, _CH), :], semo[b])

        for k in range(D - 1):
            issue(k, k)

        def body(jj, carry):
            for b in range(D):
                j = jj * D + b

                @pl.when(j + D - 1 < gch)
                def _pre():
                    issue(j + D - 1, (b + D - 1) % D)

                process(j, b)
            return carry

        lax.fori_loop(0, gch // D, body, 0)
        for t in range((gch // D) * D, gch):
            if t + D - 1 < gch:
                issue(t + D - 1, (t + D - 1) % D)
            process(t, t % D)
        for b in range(D):
            pltpu.make_async_copy(rows_a.at[b], out_hbm.at[pl.ds(0, _CH), :],
                                  semo[b]).wait()

    return gather_k(A, B, senders2d, receivers2d)


def _sc_segsum(edges_u, senders2d, receivers2d):
    from jax.experimental.pallas import tpu_sc as plsc

    mesh = plsc.VectorSubcoreMesh(core_axis_name="c", subcore_axis_name="s",
                                  num_cores=_NC, num_subcores=_NS)

    sch = senders2d.shape[1]

    @functools.partial(
        pl.kernel, mesh=mesh,
        out_type=[jax.ShapeDtypeStruct((N_NODES, HID), F32),
                  jax.ShapeDtypeStruct((N_NODES, HID), F32)],
        scratch_types=[
            pltpu.VMEM((3, _CH), jnp.int32),
            pltpu.VMEM((3, _CH, HID), F32),
            pltpu.VMEM_SHARED((N_NODES, HID), F32),
            [pltpu.SemaphoreType.DMA] * 3,
            [pltpu.SemaphoreType.DMA] * 3,
        ],
    )
    def segsum_k(eu_hbm, s_hbm, r_hbm, sent_hbm, recv_hbm,
                 idx, rows, acc, semr, semw):
        D = 3
        c = lax.axis_index("c")
        s = lax.axis_index("s")

        stage = rows.at[0]

        def zrow(r, c2):
            for k in range(HID // 16):
                stage[r, pl.ds(k * 16, 16)] = jnp.zeros((16,), F32)
            return c2

        lax.fori_loop(0, _PIECE, zrow, 0, unroll=4)
        for q in range(-(-_NPIECE // _NS)):
            p = q * _NS + s

            @pl.when(p < _NPIECE)
            def _z():
                off = pl.multiple_of(p * _PIECE, _PIECE)
                pltpu.sync_copy(stage, acc.at[pl.ds(off, _PIECE), :])

        plsc.subcore_barrier()
        ebase = pl.multiple_of(s * (sch * _CH), _CH)

        def issue(j, b):
            @pl.when(j >= D)
            def _dr():
                pltpu.make_async_copy(rows.at[b], acc.at[idx.at[b]],
                                      semw[b]).wait()

            off = pl.multiple_of(ebase + j * _CH, _CH)
            pltpu.async_copy(eu_hbm.at[pl.ds(off, _CH), :], rows.at[b],
                             semr[b])

            @pl.when(c == 0)
            def _i_s():
                pltpu.async_copy(s_hbm.at[s, j], idx.at[b], semr[b])

            @pl.when(c == 1)
            def _i_r():
                pltpu.async_copy(r_hbm.at[s, j], idx.at[b], semr[b])

        def scat(j, b):
            pltpu.make_async_copy(eu_hbm.at[pl.ds(0, _CH), :], rows.at[b],
                                  semr[b]).wait()
            pltpu.make_async_copy(s_hbm.at[0, 0], idx.at[b], semr[b]).wait()
            pltpu.async_copy(rows.at[b], acc.at[idx.at[b]], semw[b], add=True)

        for k in range(D - 1):
            issue(k, k)

        def body(jj, carry):
            for b in range(D):
                j = jj * D + b

                @pl.when(j + D - 1 < sch)
                def _pre():
                    issue(j + D - 1, (b + D - 1) % D)

                scat(j, b)
            return carry

        lax.fori_loop(0, sch // D, body, 0)
        for t in range((sch // D) * D, sch):
            if t + D - 1 < sch:
                issue(t + D - 1, (t + D - 1) % D)
            scat(t, t % D)
        for b in range(D):
            pltpu.make_async_copy(rows.at[b], acc.at[idx.at[b]],
                                  semw[b]).wait()
        plsc.subcore_barrier()

        for q in range(-(-_NPIECE // _NS)):
            p = q * _NS + s

            @pl.when(p < _NPIECE)
            def _w():
                off = pl.multiple_of(p * _PIECE, _PIECE)
                pltpu.sync_copy(acc.at[pl.ds(off, _PIECE), :], stage)

                @pl.when(c == 0)
                def _w_s():
                    pltpu.sync_copy(stage, sent_hbm.at[pl.ds(off, _PIECE), :])

                @pl.when(c == 1)
                def _w_r():
                    pltpu.sync_copy(stage, recv_hbm.at[pl.ds(off, _PIECE), :])

    return segsum_k(edges_u, senders2d, receivers2d)


def kernel(nodes, edges, W_ne, b_ne, W_ee, b_ee, W_ge, b_ge, W_e1, b_e1,
           W_e2, b_e2, W_n1, b_n1, W_n2, b_n2, W_g1, b_g1, W_g2, b_g2,
           senders, receivers, n_node, n_edge):
    n = nodes.shape[0]
    W1a = W_e1[0 * HID:1 * HID]
    W1b = W_e1[1 * HID:2 * HID]
    W1c = W_e1[2 * HID:3 * HID]
    W1d = W_e1[3 * HID:4 * HID]
    Wn1a = W_n1[0 * HID:1 * HID]
    Wn1b = W_n1[1 * HID:2 * HID]
    Wn1c = W_n1[2 * HID:3 * HID]
    Wn1d = W_n1[3 * HID:4 * HID]

    gterm = jnp.dot(b_ge.reshape(1, HID), W1d, preferred_element_type=F32)
    cn = (jnp.dot(b_ge.reshape(1, HID), Wn1d, preferred_element_type=F32)
          + b_n1.reshape(1, HID))

    E0 = 128000
    idx_g = []
    idx_s = []
    for lo, hi in ((0, E0), (E0, N_EDGES)):
        ep = hi - lo
        idx_g.append((senders[lo:hi].reshape(_NW, ep // _NW // _CH, _CH),
                      receivers[lo:hi].reshape(_NW, ep // _NW // _CH, _CH)))
        idx_s.append((senders[lo:hi].reshape(_NS, ep // _NS // _CH, _CH),
                      receivers[lo:hi].reshape(_NS, ep // _NS // _CH, _CH)))

    nodes_e, A, B = _tc_prep(nodes, W_ne, b_ne, W1b, W1c)
    G0 = _sc_gather_add(A, B, *idx_g[0])
    G1 = _sc_gather_add(A, B, *idx_g[1])
    eu0 = _tc_edge(edges, G0, 0, W_ee, W1a, b_ee, b_e1, gterm, W_e2, b_e2)
    eu1 = _tc_edge(edges, G1, E0 // _EDGE_BLK, W_ee, W1a, b_ee, b_e1, gterm,
                   W_e2, b_e2)
    sent0, recv0 = _sc_segsum(eu0, *idx_s[0])
    sent1, recv1 = _sc_segsum(eu1, *idx_s[1])
    edges_u = jnp.concatenate([eu0, eu1], axis=0)
    nodes_u, glob_u = _tc_node(nodes_e, sent0, sent1, recv0, recv1,
                               Wn1a, Wn1b, Wn1c,
                               cn, W_n2, b_n2, b_ge, W_g1, b_g1, W_g2, b_g2)
    return (nodes_u, edges_u, glob_u)


# speedup vs baseline: 1.1206x; 1.1206x over previous
import functools

import jax
import jax.numpy as jnp
from jax import lax
from jax.experimental import pallas as pl
from jax.experimental.pallas import tpu as pltpu

F32 = jnp.float32
N_NODES = 10000
N_EDGES = 320000
HID = 128
EDGE_IN = 16

_PREP_BLK = 2000


def _prep_body(nodes_ref, wne_ref, bne_ref, w1b_ref, w1c_ref,
               ne_ref, a_ref, b_ref):
    ne = jnp.dot(nodes_ref[...], wne_ref[...], preferred_element_type=F32)
    ne = ne + bne_ref[...]
    ne_ref[...] = ne
    a_ref[...] = jnp.dot(ne, w1b_ref[...], preferred_element_type=F32)
    b_ref[...] = jnp.dot(ne, w1c_ref[...], preferred_element_type=F32)


def _tc_prep(nodes, W_ne, b_ne, W1b, W1c):
    n = nodes.shape[0]
    grid = (n // _PREP_BLK,)
    blk = lambda i: (i, 0)
    full = lambda i: (0, 0)
    return pl.pallas_call(
        _prep_body,
        grid=grid,
        in_specs=[
            pl.BlockSpec((_PREP_BLK, HID), blk),
            pl.BlockSpec((HID, HID), full),
            pl.BlockSpec((1, HID), full),
            pl.BlockSpec((HID, HID), full),
            pl.BlockSpec((HID, HID), full),
        ],
        out_specs=[
            pl.BlockSpec((_PREP_BLK, HID), blk),
            pl.BlockSpec((_PREP_BLK, HID), blk),
            pl.BlockSpec((_PREP_BLK, HID), blk),
        ],
        out_shape=[jax.ShapeDtypeStruct((n, HID), F32)] * 3,
    )(nodes, W_ne, b_ne.reshape(1, HID), W1b, W1c)


_EDGE_BLK = 8000


def _edge_body(edges_ref, g_ref, wee_ref, w1a_ref, bee_ref, be1_ref,
               gterm_ref, we2_ref, be2_ref, out_ref):
    wc = jnp.dot(wee_ref[...], w1a_ref[...], preferred_element_type=F32)
    c = (jnp.dot(bee_ref[...], w1a_ref[...], preferred_element_type=F32)
         + be1_ref[...] + gterm_ref[...])
    pre = (jnp.dot(edges_ref[...], wc, preferred_element_type=F32)
           + g_ref[...] + c)
    h = jnp.maximum(pre, 0.0)
    out_ref[...] = jnp.dot(h, we2_ref[...], preferred_element_type=F32) + be2_ref[...]


def _tc_edge(edges, G, base_blk, W_ee, W1a, b_ee, b_e1, gterm, W_e2, b_e2):
    e = G.shape[0]
    grid = (e // _EDGE_BLK,)
    blk = lambda i: (i, 0)
    eblk = lambda i: (i + base_blk, 0)
    full = lambda i: (0, 0)
    return pl.pallas_call(
        _edge_body,
        grid=grid,
        in_specs=[
            pl.BlockSpec((_EDGE_BLK, EDGE_IN), eblk),
            pl.BlockSpec((_EDGE_BLK, HID), blk),
            pl.BlockSpec((EDGE_IN, HID), full),
            pl.BlockSpec((HID, HID), full),
            pl.BlockSpec((1, HID), full),
            pl.BlockSpec((1, HID), full),
            pl.BlockSpec((1, HID), full),
            pl.BlockSpec((HID, HID), full),
            pl.BlockSpec((1, HID), full),
        ],
        out_specs=pl.BlockSpec((_EDGE_BLK, HID), blk),
        out_shape=jax.ShapeDtypeStruct((e, HID), F32),
        name="edge_mlp",
    )(edges, G, W_ee, W1a, b_ee.reshape(1, HID), b_e1.reshape(1, HID),
      gterm, W_e2, b_e2.reshape(1, HID))


_NODE_BLK = 2000


def _node_body(ne_ref, sent_ref, recv_ref,
               wn1a_ref, wn1b_ref, wn1c_ref,
               cn_ref, wn2_ref, bn2_ref, bge_ref, wg1_ref, bg1_ref,
               wg2_ref, bg2_ref, nodes_u_ref, glob_ref, accn_ref, acce_ref):
    i = pl.program_id(0)
    nblk = pl.num_programs(0)
    sent = sent_ref[...]
    recv = recv_ref[...]
    pre = (jnp.dot(ne_ref[...], wn1a_ref[...], preferred_element_type=F32)
           + jnp.dot(sent, wn1b_ref[...], preferred_element_type=F32)
           + jnp.dot(recv, wn1c_ref[...], preferred_element_type=F32)
           + cn_ref[...])
    nu = (jnp.dot(jnp.maximum(pre, 0.0), wn2_ref[...], preferred_element_type=F32)
          + bn2_ref[...])
    nodes_u_ref[...] = nu
    blk_n = jnp.sum(nu, axis=0, keepdims=True)
    blk_e = jnp.sum(sent, axis=0, keepdims=True)

    @pl.when(i == 0)
    def _init():
        accn_ref[...] = blk_n
        acce_ref[...] = blk_e

    @pl.when(i > 0)
    def _acc():
        accn_ref[...] += blk_n
        acce_ref[...] += blk_e

    @pl.when(i == nblk - 1)
    def _glob():
        gf = jnp.concatenate([accn_ref[...], acce_ref[...], bge_ref[...]], axis=1)
        gh = jnp.maximum(
            jnp.dot(gf, wg1_ref[...], preferred_element_type=F32) + bg1_ref[...], 0.0)
        glob_ref[...] = (jnp.dot(gh, wg2_ref[...], preferred_element_type=F32)
                         + bg2_ref[...])


def _tc_node(nodes_e, sent_agg, recv_agg,
             Wn1a, Wn1b, Wn1c, cn, W_n2, b_n2,
             b_ge, W_g1, b_g1, W_g2, b_g2):
    n = nodes_e.shape[0]
    grid = (n // _NODE_BLK,)
    blk = lambda i: (i, 0)
    full = lambda i: (0, 0)
    return pl.pallas_call(
        _node_body,
        grid=grid,
        in_specs=[
            pl.BlockSpec((_NODE_BLK, HID), blk),
            pl.BlockSpec((_NODE_BLK, HID), blk),
            pl.BlockSpec((_NODE_BLK, HID), blk),
            pl.BlockSpec((HID, HID), full),
            pl.BlockSpec((HID, HID), full),
            pl.BlockSpec((HID, HID), full),
            pl.BlockSpec((1, HID), full),
            pl.BlockSpec((HID, HID), full),
            pl.BlockSpec((1, HID), full),
            pl.BlockSpec((1, HID), full),
            pl.BlockSpec((3 * HID, HID), full),
            pl.BlockSpec((1, HID), full),
            pl.BlockSpec((HID, 3), full),
            pl.BlockSpec((1, 3), full),
        ],
        out_specs=[
            pl.BlockSpec((_NODE_BLK, HID), blk),
            pl.BlockSpec((1, 3), full),
        ],
        out_shape=[
            jax.ShapeDtypeStruct((n, HID), F32),
            jax.ShapeDtypeStruct((1, 3), F32),
        ],
        scratch_shapes=[pltpu.VMEM((1, HID), F32), pltpu.VMEM((1, HID), F32)],
    )(nodes_e, sent_agg, recv_agg, Wn1a, Wn1b, Wn1c, cn, W_n2,
      b_n2.reshape(1, HID), b_ge.reshape(1, HID), W_g1, b_g1.reshape(1, HID),
      W_g2, b_g2.reshape(1, 3))


_NC = 2
_NS = 16
_NW = _NC * _NS
_CH = 80
_GCH = N_EDGES // _NW // _CH
_SCH = N_EDGES // _NS // _CH
_PIECE = 80
_NPIECE = N_NODES // _PIECE


def _sc_gather_add(A, B, senders2d, receivers2d):
    from jax.experimental.pallas import tpu_sc as plsc

    mesh = plsc.VectorSubcoreMesh(core_axis_name="c", subcore_axis_name="s",
                                  num_cores=_NC, num_subcores=_NS)

    D = 4
    gch = senders2d.shape[1]

    @functools.partial(
        pl.kernel, mesh=mesh,
        out_type=jax.ShapeDtypeStruct((_NW * gch * _CH, HID), F32),
        scratch_types=[
            pltpu.VMEM((gch, _CH), jnp.int32),
            pltpu.VMEM((gch, _CH), jnp.int32),
            pltpu.VMEM((D, _CH, HID), F32),
            pltpu.VMEM((D, _CH, HID), F32),
            [pltpu.SemaphoreType.DMA] * D,
            [pltpu.SemaphoreType.DMA] * D,
        ],
    )
    def gather_k(a_hbm, b_hbm, s_hbm, r_hbm, out_hbm,
                 sidx, ridx, rows_a, rows_b, semg, semo):
        wid = lax.axis_index("s") * _NC + lax.axis_index("c")
        pltpu.sync_copy(s_hbm.at[wid], sidx)
        pltpu.sync_copy(r_hbm.at[wid], ridx)
        ebase = pl.multiple_of(wid * (gch * _CH), _CH)

        def issue(j, b):
            @pl.when(j >= D)
            def _dr():
                pltpu.make_async_copy(rows_a.at[b],
                                      out_hbm.at[pl.ds(0, _CH), :],
                                      semo[b]).wait()

            pltpu.async_copy(a_hbm.at[sidx.at[j]], rows_a.at[b], semg[b])
            pltpu.async_copy(b_hbm.at[ridx.at[j]], rows_b.at[b], semg[b])

        def process(j, b):
            pltpu.make_async_copy(a_hbm.at[sidx.at[0]], rows_a.at[b],
                                  semg[b]).wait()
            pltpu.make_async_copy(b_hbm.at[ridx.at[0]], rows_b.at[b],
                                  semg[b]).wait()
            ra = rows_a.at[b]
            rb = rows_b.at[b]

            def addrow(r, c2):
                for k in range(HID // 16):
                    plsc.addupdate(ra.at[r, pl.ds(k * 16, 16)],
                                   rb[r, pl.ds(k * 16, 16)])
                return c2

            lax.fori_loop(0, _CH, addrow, 0, unroll=4)
            off = pl.multiple_of(ebase + j * _CH, _CH)
            pltpu.async_copy(ra, out_hbm.at[pl.ds(off, _CH), :], semo[b])

        for k in range(D - 1):
            issue(k, k)

        def body(jj, carry):
            for b in range(D):
                j = jj * D + b

                @pl.when(j + D - 1 < gch)
                def _pre():
                    issue(j + D - 1, (b + D - 1) % D)

                process(j, b)
            return carry

        lax.fori_loop(0, gch // D, body, 0)
        for t in range((gch // D) * D, gch):
            if t + D - 1 < gch:
                issue(t + D - 1, (t + D - 1) % D)
            process(t, t % D)
        for b in range(D):
            pltpu.make_async_copy(rows_a.at[b], out_hbm.at[pl.ds(0, _CH), :],
                                  semo[b]).wait()

    return gather_k(A, B, senders2d, receivers2d)


def _sc_segsum(edges_u, senders2d, receivers2d):
    from jax.experimental.pallas import tpu_sc as plsc

    mesh = plsc.VectorSubcoreMesh(core_axis_name="c", subcore_axis_name="s",
                                  num_cores=_NC, num_subcores=_NS)

    sch = senders2d.shape[1]

    @functools.partial(
        pl.kernel, mesh=mesh,
        out_type=[jax.ShapeDtypeStruct((N_NODES, HID), F32),
                  jax.ShapeDtypeStruct((N_NODES, HID), F32)],
        scratch_types=[
            pltpu.VMEM((3, _CH), jnp.int32),
            pltpu.VMEM((3, _CH, HID), F32),
            pltpu.VMEM_SHARED((N_NODES, HID), F32),
            [pltpu.SemaphoreType.DMA] * 3,
            [pltpu.SemaphoreType.DMA] * 3,
        ],
    )
    def segsum_k(eu_hbm, s_hbm, r_hbm, sent_hbm, recv_hbm,
                 idx, rows, acc, semr, semw):
        D = 3
        c = lax.axis_index("c")
        s = lax.axis_index("s")

        stage = rows.at[0]

        def zrow(r, c2):
            for k in range(HID // 16):
                stage[r, pl.ds(k * 16, 16)] = jnp.zeros((16,), F32)
            return c2

        lax.fori_loop(0, _PIECE, zrow, 0, unroll=4)
        nq = -(-_NPIECE // _NS)
        for q in range(nq):
            p = q * _NS + s

            @pl.when(p < _NPIECE)
            def _z():
                off = pl.multiple_of(p * _PIECE, _PIECE)
                pltpu.async_copy(stage, acc.at[pl.ds(off, _PIECE), :],
                                 semr[0])

        for q in range(nq):
            p = q * _NS + s

            @pl.when(p < _NPIECE)
            def _zd():
                pltpu.make_async_copy(stage, acc.at[pl.ds(0, _PIECE), :],
                                      semr[0]).wait()

        plsc.subcore_barrier()
        ebase = pl.multiple_of(s * (sch * _CH), _CH)

        def issue(j, b):
            @pl.when(j >= D)
            def _dr():
                pltpu.make_async_copy(rows.at[b], acc.at[idx.at[b]],
                                      semw[b]).wait()

            off = pl.multiple_of(ebase + j * _CH, _CH)
            pltpu.async_copy(eu_hbm.at[pl.ds(off, _CH), :], rows.at[b],
                             semr[b])

            @pl.when(c == 0)
            def _i_s():
                pltpu.async_copy(s_hbm.at[s, j], idx.at[b], semr[b])

            @pl.when(c == 1)
            def _i_r():
                pltpu.async_copy(r_hbm.at[s, j], idx.at[b], semr[b])

        def scat(j, b):
            pltpu.make_async_copy(eu_hbm.at[pl.ds(0, _CH), :], rows.at[b],
                                  semr[b]).wait()
            pltpu.make_async_copy(s_hbm.at[0, 0], idx.at[b], semr[b]).wait()
            pltpu.async_copy(rows.at[b], acc.at[idx.at[b]], semw[b], add=True)

        for k in range(D - 1):
            issue(k, k)

        def body(jj, carry):
            for b in range(D):
                j = jj * D + b

                @pl.when(j + D - 1 < sch)
                def _pre():
                    issue(j + D - 1, (b + D - 1) % D)

                scat(j, b)
            return carry

        lax.fori_loop(0, sch // D, body, 0)
        for t in range((sch // D) * D, sch):
            if t + D - 1 < sch:
                issue(t + D - 1, (t + D - 1) % D)
            scat(t, t % D)
        for b in range(D):
            pltpu.make_async_copy(rows.at[b], acc.at[idx.at[b]],
                                  semw[b]).wait()
        plsc.subcore_barrier()

        for q in range(nq):
            p = q * _NS + s
            b = q % D

            @pl.when(p < _NPIECE)
            def _w():
                if q >= D:
                    pltpu.make_async_copy(rows.at[b],
                                          sent_hbm.at[pl.ds(0, _PIECE), :],
                                          semw[b]).wait()
                off = pl.multiple_of(p * _PIECE, _PIECE)
                pltpu.sync_copy(acc.at[pl.ds(off, _PIECE), :], rows.at[b])

                @pl.when(c == 0)
                def _w_s():
                    pltpu.async_copy(rows.at[b],
                                     sent_hbm.at[pl.ds(off, _PIECE), :],
                                     semw[b])

                @pl.when(c == 1)
                def _w_r():
                    pltpu.async_copy(rows.at[b],
                                     recv_hbm.at[pl.ds(off, _PIECE), :],
                                     semw[b])

        for b in range(D):
            pltpu.make_async_copy(rows.at[b], sent_hbm.at[pl.ds(0, _PIECE), :],
                                  semw[b]).wait()

    return segsum_k(edges_u, senders2d, receivers2d)


def kernel(nodes, edges, W_ne, b_ne, W_ee, b_ee, W_ge, b_ge, W_e1, b_e1,
           W_e2, b_e2, W_n1, b_n1, W_n2, b_n2, W_g1, b_g1, W_g2, b_g2,
           senders, receivers, n_node, n_edge):
    n = nodes.shape[0]
    W1a = W_e1[0 * HID:1 * HID]
    W1b = W_e1[1 * HID:2 * HID]
    W1c = W_e1[2 * HID:3 * HID]
    W1d = W_e1[3 * HID:4 * HID]
    Wn1a = W_n1[0 * HID:1 * HID]
    Wn1b = W_n1[1 * HID:2 * HID]
    Wn1c = W_n1[2 * HID:3 * HID]
    Wn1d = W_n1[3 * HID:4 * HID]

    gterm = jnp.dot(b_ge.reshape(1, HID), W1d, preferred_element_type=F32)
    cn = (jnp.dot(b_ge.reshape(1, HID), Wn1d, preferred_element_type=F32)
          + b_n1.reshape(1, HID))

    senders_g = senders.reshape(_NW, _GCH, _CH)
    receivers_g = receivers.reshape(_NW, _GCH, _CH)
    senders_s = senders.reshape(_NS, _SCH, _CH)
    receivers_s = receivers.reshape(_NS, _SCH, _CH)

    nodes_e, A, B = _tc_prep(nodes, W_ne, b_ne, W1b, W1c)
    G = _sc_gather_add(A, B, senders_g, receivers_g)
    edges_u = _tc_edge(edges, G, 0, W_ee, W1a, b_ee, b_e1, gterm, W_e2, b_e2)
    sent_agg, recv_agg = _sc_segsum(edges_u, senders_s, receivers_s)
    nodes_u, glob_u = _tc_node(nodes_e, sent_agg, recv_agg,
                               Wn1a, Wn1b, Wn1c,
                               cn, W_n2, b_n2, b_ge, W_g1, b_g1, W_g2, b_g2)
    return (nodes_u, edges_u, glob_u)


# speedup vs baseline: 1.1239x; 1.0030x over previous
import functools

import jax
import jax.numpy as jnp
from jax import lax
from jax.experimental import pallas as pl
from jax.experimental.pallas import tpu as pltpu

F32 = jnp.float32
N_NODES = 10000
N_EDGES = 320000
HID = 128
EDGE_IN = 16

_PREP_BLK = 2000


def _prep_body(nodes_ref, wne_ref, bne_ref, w1b_ref, w1c_ref,
               ne_ref, a_ref, b_ref):
    ne = jnp.dot(nodes_ref[...], wne_ref[...], preferred_element_type=F32)
    ne = ne + bne_ref[...]
    ne_ref[...] = ne
    a_ref[...] = jnp.dot(ne, w1b_ref[...], preferred_element_type=F32)
    b_ref[...] = jnp.dot(ne, w1c_ref[...], preferred_element_type=F32)


def _tc_prep(nodes, W_ne, b_ne, W1b, W1c):
    n = nodes.shape[0]
    grid = (n // _PREP_BLK,)
    blk = lambda i: (i, 0)
    full = lambda i: (0, 0)
    return pl.pallas_call(
        _prep_body,
        grid=grid,
        in_specs=[
            pl.BlockSpec((_PREP_BLK, HID), blk),
            pl.BlockSpec((HID, HID), full),
            pl.BlockSpec((1, HID), full),
            pl.BlockSpec((HID, HID), full),
            pl.BlockSpec((HID, HID), full),
        ],
        out_specs=[
            pl.BlockSpec((_PREP_BLK, HID), blk),
            pl.BlockSpec((_PREP_BLK, HID), blk),
            pl.BlockSpec((_PREP_BLK, HID), blk),
        ],
        out_shape=[jax.ShapeDtypeStruct((n, HID), F32)] * 3,
    )(nodes, W_ne, b_ne.reshape(1, HID), W1b, W1c)


_EDGE_BLK = 16000


def _edge_body(edges_ref, g_ref, wee_ref, w1a_ref, bee_ref, be1_ref,
               gterm_ref, we2_ref, be2_ref, out_ref):
    wc = jnp.dot(wee_ref[...], w1a_ref[...], preferred_element_type=F32)
    c = (jnp.dot(bee_ref[...], w1a_ref[...], preferred_element_type=F32)
         + be1_ref[...] + gterm_ref[...])
    pre = (jnp.dot(edges_ref[...], wc, preferred_element_type=F32)
           + g_ref[...] + c)
    h = jnp.maximum(pre, 0.0)
    out_ref[...] = jnp.dot(h, we2_ref[...], preferred_element_type=F32) + be2_ref[...]


def _tc_edge(edges, G, base_blk, W_ee, W1a, b_ee, b_e1, gterm, W_e2, b_e2):
    e = G.shape[0]
    grid = (e // _EDGE_BLK,)
    blk = lambda i: (i, 0)
    eblk = lambda i: (i + base_blk, 0)
    full = lambda i: (0, 0)
    return pl.pallas_call(
        _edge_body,
        grid=grid,
        in_specs=[
            pl.BlockSpec((_EDGE_BLK, EDGE_IN), eblk),
            pl.BlockSpec((_EDGE_BLK, HID), blk),
            pl.BlockSpec((EDGE_IN, HID), full),
            pl.BlockSpec((HID, HID), full),
            pl.BlockSpec((1, HID), full),
            pl.BlockSpec((1, HID), full),
            pl.BlockSpec((1, HID), full),
            pl.BlockSpec((HID, HID), full),
            pl.BlockSpec((1, HID), full),
        ],
        out_specs=pl.BlockSpec((_EDGE_BLK, HID), blk),
        out_shape=jax.ShapeDtypeStruct((e, HID), F32),
        name="edge_mlp",
    )(edges, G, W_ee, W1a, b_ee.reshape(1, HID), b_e1.reshape(1, HID),
      gterm, W_e2, b_e2.reshape(1, HID))


_NODE_BLK = 2000


def _node_body(ne_ref, sent_ref, recv_ref,
               wn1a_ref, wn1b_ref, wn1c_ref,
               cn_ref, wn2_ref, bn2_ref, bge_ref, wg1_ref, bg1_ref,
               wg2_ref, bg2_ref, nodes_u_ref, glob_ref, accn_ref, acce_ref):
    i = pl.program_id(0)
    nblk = pl.num_programs(0)
    sent = sent_ref[...]
    recv = recv_ref[...]
    pre = (jnp.dot(ne_ref[...], wn1a_ref[...], preferred_element_type=F32)
           + jnp.dot(sent, wn1b_ref[...], preferred_element_type=F32)
           + jnp.dot(recv, wn1c_ref[...], preferred_element_type=F32)
           + cn_ref[...])
    nu = (jnp.dot(jnp.maximum(pre, 0.0), wn2_ref[...], preferred_element_type=F32)
          + bn2_ref[...])
    nodes_u_ref[...] = nu
    blk_n = jnp.sum(nu, axis=0, keepdims=True)
    blk_e = jnp.sum(sent, axis=0, keepdims=True)

    @pl.when(i == 0)
    def _init():
        accn_ref[...] = blk_n
        acce_ref[...] = blk_e

    @pl.when(i > 0)
    def _acc():
        accn_ref[...] += blk_n
        acce_ref[...] += blk_e

    @pl.when(i == nblk - 1)
    def _glob():
        gf = jnp.concatenate([accn_ref[...], acce_ref[...], bge_ref[...]], axis=1)
        gh = jnp.maximum(
            jnp.dot(gf, wg1_ref[...], preferred_element_type=F32) + bg1_ref[...], 0.0)
        glob_ref[...] = (jnp.dot(gh, wg2_ref[...], preferred_element_type=F32)
                         + bg2_ref[...])


def _tc_node(nodes_e, sent_agg, recv_agg,
             Wn1a, Wn1b, Wn1c, cn, W_n2, b_n2,
             b_ge, W_g1, b_g1, W_g2, b_g2):
    n = nodes_e.shape[0]
    grid = (n // _NODE_BLK,)
    blk = lambda i: (i, 0)
    full = lambda i: (0, 0)
    return pl.pallas_call(
        _node_body,
        grid=grid,
        in_specs=[
            pl.BlockSpec((_NODE_BLK, HID), blk),
            pl.BlockSpec((_NODE_BLK, HID), blk),
            pl.BlockSpec((_NODE_BLK, HID), blk),
            pl.BlockSpec((HID, HID), full),
            pl.BlockSpec((HID, HID), full),
            pl.BlockSpec((HID, HID), full),
            pl.BlockSpec((1, HID), full),
            pl.BlockSpec((HID, HID), full),
            pl.BlockSpec((1, HID), full),
            pl.BlockSpec((1, HID), full),
            pl.BlockSpec((3 * HID, HID), full),
            pl.BlockSpec((1, HID), full),
            pl.BlockSpec((HID, 3), full),
            pl.BlockSpec((1, 3), full),
        ],
        out_specs=[
            pl.BlockSpec((_NODE_BLK, HID), blk),
            pl.BlockSpec((1, 3), full),
        ],
        out_shape=[
            jax.ShapeDtypeStruct((n, HID), F32),
            jax.ShapeDtypeStruct((1, 3), F32),
        ],
        scratch_shapes=[pltpu.VMEM((1, HID), F32), pltpu.VMEM((1, HID), F32)],
    )(nodes_e, sent_agg, recv_agg, Wn1a, Wn1b, Wn1c, cn, W_n2,
      b_n2.reshape(1, HID), b_ge.reshape(1, HID), W_g1, b_g1.reshape(1, HID),
      W_g2, b_g2.reshape(1, 3))


_NC = 2
_NS = 16
_NW = _NC * _NS
_CH = 80
_GCH = N_EDGES // _NW // _CH
_SCH = N_EDGES // _NS // _CH
_PIECE = 80
_NPIECE = N_NODES // _PIECE


def _sc_gather_add(A, B, senders2d, receivers2d):
    from jax.experimental.pallas import tpu_sc as plsc

    mesh = plsc.VectorSubcoreMesh(core_axis_name="c", subcore_axis_name="s",
                                  num_cores=_NC, num_subcores=_NS)

    D = 4
    gch = senders2d.shape[1]

    @functools.partial(
        pl.kernel, mesh=mesh,
        out_type=jax.ShapeDtypeStruct((_NW * gch * _CH, HID), F32),
        scratch_types=[
            pltpu.VMEM((gch, _CH), jnp.int32),
            pltpu.VMEM((gch, _CH), jnp.int32),
            pltpu.VMEM((D, _CH, HID), F32),
            pltpu.VMEM((D, _CH, HID), F32),
            [pltpu.SemaphoreType.DMA] * D,
            [pltpu.SemaphoreType.DMA] * D,
        ],
    )
    def gather_k(a_hbm, b_hbm, s_hbm, r_hbm, out_hbm,
                 sidx, ridx, rows_a, rows_b, semg, semo):
        wid = lax.axis_index("s") * _NC + lax.axis_index("c")
        pltpu.sync_copy(s_hbm.at[wid], sidx)
        pltpu.sync_copy(r_hbm.at[wid], ridx)
        ebase = pl.multiple_of(wid * (gch * _CH), _CH)

        def issue(j, b):
            @pl.when(j >= D)
            def _dr():
                pltpu.make_async_copy(rows_a.at[b],
                                      out_hbm.at[pl.ds(0, _CH), :],
                                      semo[b]).wait()

            pltpu.async_copy(a_hbm.at[sidx.at[j]], rows_a.at[b], semg[b])
            pltpu.async_copy(b_hbm.at[ridx.at[j]], rows_b.at[b], semg[b])

        def process(j, b):
            pltpu.make_async_copy(a_hbm.at[sidx.at[0]], rows_a.at[b],
                                  semg[b]).wait()
            pltpu.make_async_copy(b_hbm.at[ridx.at[0]], rows_b.at[b],
                                  semg[b]).wait()
            ra = rows_a.at[b]
            rb = rows_b.at[b]

            def addrow(r, c2):
                for k in range(HID // 16):
                    plsc.addupdate(ra.at[r, pl.ds(k * 16, 16)],
                                   rb[r, pl.ds(k * 16, 16)])
                return c2

            lax.fori_loop(0, _CH, addrow, 0, unroll=4)
            off = pl.multiple_of(ebase + j * _CH, _CH)
            pltpu.async_copy(ra, out_hbm.at[pl.ds(off, _CH), :], semo[b])

        for k in range(D - 1):
            issue(k, k)

        def body(jj, carry):
            for b in range(D):
                j = jj * D + b

                @pl.when(j + D - 1 < gch)
                def _pre():
                    issue(j + D - 1, (b + D - 1) % D)

                process(j, b)
            return carry

        lax.fori_loop(0, gch // D, body, 0)
        for t in range((gch // D) * D, gch):
            if t + D - 1 < gch:
                issue(t + D - 1, (t + D - 1) % D)
            process(t, t % D)
        for b in range(D):
            pltpu.make_async_copy(rows_a.at[b], out_hbm.at[pl.ds(0, _CH), :],
                                  semo[b]).wait()

    return gather_k(A, B, senders2d, receivers2d)


def _sc_segsum(edges_u, senders2d, receivers2d):
    from jax.experimental.pallas import tpu_sc as plsc

    mesh = plsc.VectorSubcoreMesh(core_axis_name="c", subcore_axis_name="s",
                                  num_cores=_NC, num_subcores=_NS)

    sch = senders2d.shape[1]

    @functools.partial(
        pl.kernel, mesh=mesh,
        out_type=[jax.ShapeDtypeStruct((N_NODES, HID), F32),
                  jax.ShapeDtypeStruct((N_NODES, HID), F32)],
        scratch_types=[
            pltpu.VMEM((3, _CH), jnp.int32),
            pltpu.VMEM((3, _CH, HID), F32),
            pltpu.VMEM_SHARED((N_NODES, HID), F32),
            [pltpu.SemaphoreType.DMA] * 3,
            [pltpu.SemaphoreType.DMA] * 3,
        ],
    )
    def segsum_k(eu_hbm, s_hbm, r_hbm, sent_hbm, recv_hbm,
                 idx, rows, acc, semr, semw):
        D = 3
        c = lax.axis_index("c")
        s = lax.axis_index("s")

        stage = rows.at[0]

        def zrow(r, c2):
            for k in range(HID // 16):
                stage[r, pl.ds(k * 16, 16)] = jnp.zeros((16,), F32)
            return c2

        lax.fori_loop(0, _PIECE, zrow, 0, unroll=4)
        nq = -(-_NPIECE // _NS)
        for q in range(nq):
            p = q * _NS + s

            @pl.when(p < _NPIECE)
            def _z():
                off = pl.multiple_of(p * _PIECE, _PIECE)
                pltpu.async_copy(stage, acc.at[pl.ds(off, _PIECE), :],
                                 semr[0])

        for q in range(nq):
            p = q * _NS + s

            @pl.when(p < _NPIECE)
            def _zd():
                pltpu.make_async_copy(stage, acc.at[pl.ds(0, _PIECE), :],
                                      semr[0]).wait()

        plsc.subcore_barrier()
        ebase = pl.multiple_of(s * (sch * _CH), _CH)

        def issue(j, b):
            @pl.when(j >= D)
            def _dr():
                pltpu.make_async_copy(rows.at[b], acc.at[idx.at[b]],
                                      semw[b]).wait()

            off = pl.multiple_of(ebase + j * _CH, _CH)
            pltpu.async_copy(eu_hbm.at[pl.ds(off, _CH), :], rows.at[b],
                             semr[b])

            @pl.when(c == 0)
            def _i_s():
                pltpu.async_copy(s_hbm.at[s, j], idx.at[b], semr[b])

            @pl.when(c == 1)
            def _i_r():
                pltpu.async_copy(r_hbm.at[s, j], idx.at[b], semr[b])

        def scat(j, b):
            pltpu.make_async_copy(eu_hbm.at[pl.ds(0, _CH), :], rows.at[b],
                                  semr[b]).wait()
            pltpu.make_async_copy(s_hbm.at[0, 0], idx.at[b], semr[b]).wait()
            pltpu.async_copy(rows.at[b], acc.at[idx.at[b]], semw[b], add=True)

        for k in range(D - 1):
            issue(k, k)

        def body(jj, carry):
            for b in range(D):
                j = jj * D + b

                @pl.when(j + D - 1 < sch)
                def _pre():
                    issue(j + D - 1, (b + D - 1) % D)

                scat(j, b)
            return carry

        lax.fori_loop(0, sch // D, body, 0)
        for t in range((sch // D) * D, sch):
            if t + D - 1 < sch:
                issue(t + D - 1, (t + D - 1) % D)
            scat(t, t % D)
        for b in range(D):
            pltpu.make_async_copy(rows.at[b], acc.at[idx.at[b]],
                                  semw[b]).wait()
        plsc.subcore_barrier()

        for q in range(nq):
            p = q * _NS + s
            b = q % D

            @pl.when(p < _NPIECE)
            def _w():
                if q >= D:
                    pltpu.make_async_copy(rows.at[b],
                                          sent_hbm.at[pl.ds(0, _PIECE), :],
                                          semw[b]).wait()
                off = pl.multiple_of(p * _PIECE, _PIECE)
                pltpu.sync_copy(acc.at[pl.ds(off, _PIECE), :], rows.at[b])

                @pl.when(c == 0)
                def _w_s():
                    pltpu.async_copy(rows.at[b],
                                     sent_hbm.at[pl.ds(off, _PIECE), :],
                                     semw[b])

                @pl.when(c == 1)
                def _w_r():
                    pltpu.async_copy(rows.at[b],
                                     recv_hbm.at[pl.ds(off, _PIECE), :],
                                     semw[b])

        for b in range(D):
            pltpu.make_async_copy(rows.at[b], sent_hbm.at[pl.ds(0, _PIECE), :],
                                  semw[b]).wait()

    return segsum_k(edges_u, senders2d, receivers2d)


def kernel(nodes, edges, W_ne, b_ne, W_ee, b_ee, W_ge, b_ge, W_e1, b_e1,
           W_e2, b_e2, W_n1, b_n1, W_n2, b_n2, W_g1, b_g1, W_g2, b_g2,
           senders, receivers, n_node, n_edge):
    n = nodes.shape[0]
    W1a = W_e1[0 * HID:1 * HID]
    W1b = W_e1[1 * HID:2 * HID]
    W1c = W_e1[2 * HID:3 * HID]
    W1d = W_e1[3 * HID:4 * HID]
    Wn1a = W_n1[0 * HID:1 * HID]
    Wn1b = W_n1[1 * HID:2 * HID]
    Wn1c = W_n1[2 * HID:3 * HID]
    Wn1d = W_n1[3 * HID:4 * HID]

    gterm = jnp.dot(b_ge.reshape(1, HID), W1d, preferred_element_type=F32)
    cn = (jnp.dot(b_ge.reshape(1, HID), Wn1d, preferred_element_type=F32)
          + b_n1.reshape(1, HID))

    senders_g = senders.reshape(_NW, _GCH, _CH)
    receivers_g = receivers.reshape(_NW, _GCH, _CH)
    senders_s = senders.reshape(_NS, _SCH, _CH)
    receivers_s = receivers.reshape(_NS, _SCH, _CH)

    nodes_e, A, B = _tc_prep(nodes, W_ne, b_ne, W1b, W1c)
    G = _sc_gather_add(A, B, senders_g, receivers_g)
    edges_u = _tc_edge(edges, G, 0, W_ee, W1a, b_ee, b_e1, gterm, W_e2, b_e2)
    sent_agg, recv_agg = _sc_segsum(edges_u, senders_s, receivers_s)
    nodes_u, glob_u = _tc_node(nodes_e, sent_agg, recv_agg,
                               Wn1a, Wn1b, Wn1c,
                               cn, W_n2, b_n2, b_ge, W_g1, b_g1, W_g2, b_g2)
    return (nodes_u, edges_u, glob_u)
